# baseline (speedup 1.0000x reference)
import functools

import jax
import jax.numpy as jnp
from jax import lax
from jax.experimental import pallas as pl
from jax.experimental.pallas import tpu as pltpu
from jax.experimental.pallas import tpu_sc as plsc

E = 8
K = 2
H = 1024
F = 4096
T = 2048
BT = 256
NB = (T * K) // BT + E
P = NB * BT
A = T * K

NC = 2
NS = 16
NW = NC * NS


def _routing_body(logits_ref, pos_ref, g0b_ref, g1b_ref, meta_ref, oh_ref,
                  pre_ref):
    lg = logits_ref[...]
    lane = lax.broadcasted_iota(jnp.int32, (T, E), 1)
    m1 = jnp.max(lg, axis=1, keepdims=True)
    i1 = jnp.min(jnp.where(lg == m1, lane, E), axis=1, keepdims=True)
    lg2 = jnp.where(lane == i1, -jnp.inf, lg)
    m2 = jnp.max(lg2, axis=1, keepdims=True)
    i2 = jnp.min(jnp.where(lg2 == m2, lane, E), axis=1, keepdims=True)
    t = jnp.exp(m2 - m1)
    g1 = 1.0 / (1.0 + t)
    g2 = t / (1.0 + t)

    sel1 = (lane == i1)
    sel2 = (lane == i2)
    oh = jnp.where(sel1 | sel2, 1.0, 0.0)
    oh_ref[...] = oh

    C = 512
    r = lax.broadcasted_iota(jnp.int32, (C, C), 0)
    c = lax.broadcasted_iota(jnp.int32, (C, C), 1)
    tri = jnp.where(r > c, 1.0, 0.0).astype(jnp.bfloat16)

    def chunk(ci, carry):
        ohc = oh_ref[pl.ds(ci * C, C), :]
        excl = jnp.dot(tri, ohc.astype(jnp.bfloat16),
                       preferred_element_type=jnp.float32)
        pre_ref[pl.ds(ci * C, C), :] = excl + carry
        return carry + jnp.sum(ohc, axis=0, keepdims=True)

    counts = lax.fori_loop(0, T // C, chunk, jnp.zeros((1, E), jnp.float32))

    nb = jnp.floor((counts + (BT - 1)) * (1.0 / BT))
    ur = lax.broadcasted_iota(jnp.int32, (E, E), 0)
    uc = lax.broadcasted_iota(jnp.int32, (E, E), 1)
    umat = jnp.where(ur < uc, 1.0, 0.0)
    pstart = jnp.dot(nb, umat, preferred_element_type=jnp.float32) * BT

    pre = pre_ref[...] + pstart
    pos1 = jnp.sum(jnp.where(sel1, pre, 0.0), axis=1, keepdims=True)
    pos2 = jnp.sum(jnp.where(sel2, pre, 0.0), axis=1, keepdims=True)
    pos_ref[...] = jnp.concatenate([pos1, pos2], axis=1).astype(jnp.int32)
    g0b_ref[...] = jnp.broadcast_to(g1, (T, 16))
    g1b_ref[...] = jnp.broadcast_to(g2, (T, 16))

    meta = jnp.concatenate([counts, nb, jnp.zeros((E - 2, E), jnp.float32)],
                           axis=0)
    meta_ref[...] = meta


def _routing(router_logits):
    return pl.pallas_call(
        _routing_body,
        out_shape=[
            jax.ShapeDtypeStruct((T, K), jnp.int32),
            jax.ShapeDtypeStruct((T, 16), jnp.float32),
            jax.ShapeDtypeStruct((T, 16), jnp.float32),
            jax.ShapeDtypeStruct((E, E), jnp.float32),
        ],
        in_specs=[pl.BlockSpec((T, E), lambda: (0, 0))],
        out_specs=[
            pl.BlockSpec((T, K), lambda: (0, 0)),
            pl.BlockSpec((T, 16), lambda: (0, 0)),
            pl.BlockSpec((T, 16), lambda: (0, 0)),
            pl.BlockSpec((E, E), lambda: (0, 0)),
        ],
        scratch_shapes=[
            pltpu.VMEM((T, E), jnp.float32),
            pltpu.VMEM((T, E), jnp.float32),
        ],
    )(router_logits)


_APW = A // NW
_RPW = P // NW
_GCH = _RPW // 2
_TPW = T // NW
_CCH = _TPW // 4

_sc_cache = {}


def _sc_kernels():
    if "k" in _sc_cache:
        return _sc_cache["k"]
    mesh = plsc.VectorSubcoreMesh(core_axis_name="c", subcore_axis_name="s")

    @functools.partial(
        pl.kernel, mesh=mesh,
        out_type=jax.ShapeDtypeStruct((P, H // 2), jnp.int32),
        scratch_types=[
            pltpu.VMEM((_TPW,), jnp.int32),
            pltpu.VMEM((_TPW,), jnp.int32),
            pltpu.VMEM((_TPW, H // 2), jnp.int32),
            pltpu.SemaphoreType.DMA,
        ],
    )
    def sc_dispatch(pos0_hbm, pos1_hbm, x_hbm, xs_out, p0_v, p1_v, rows_v,
                    sem):
        wid = lax.axis_index("s") * NC + lax.axis_index("c")
        base = wid * _TPW
        pltpu.sync_copy(x_hbm.at[pl.ds(base, _TPW)], rows_v)
        pltpu.sync_copy(pos0_hbm.at[pl.ds(base, _TPW)], p0_v)
        pltpu.sync_copy(pos1_hbm.at[pl.ds(base, _TPW)], p1_v)
        c0 = pltpu.async_copy(rows_v, xs_out.at[p0_v], sem)
        c1 = pltpu.async_copy(rows_v, xs_out.at[p1_v], sem)
        c0.wait()
        c1.wait()

    @functools.partial(
        pl.kernel, mesh=mesh,
        out_type=jax.ShapeDtypeStruct((T, H), jnp.float32),
        scratch_types=[
            pltpu.VMEM((_CCH,), jnp.int32),
            pltpu.VMEM((_CCH,), jnp.int32),
            pltpu.VMEM((_CCH, 16), jnp.float32),
            pltpu.VMEM((_CCH, 16), jnp.float32),
            pltpu.VMEM((_CCH, 2 * H), jnp.float32),
            pltpu.VMEM((_CCH, 2 * H), jnp.float32),
            pltpu.VMEM((_CCH, H), jnp.float32),
            pltpu.SemaphoreType.DMA,
        ],
    )
    def sc_combine(pos0_hbm, pos1_hbm, g0b_hbm, g1b_hbm, y_hbm, out_hbm,
                   p0_v, p1_v, g0_v, g1_v, a_v, b_v, o_v, sem):
        wid = lax.axis_index("s") * NC + lax.axis_index("c")
        for ch in range(_TPW // _CCH):
            tbase = wid * _TPW + ch * _CCH
            pltpu.sync_copy(pos0_hbm.at[pl.ds(tbase, _CCH)], p0_v)
            pltpu.sync_copy(pos1_hbm.at[pl.ds(tbase, _CCH)], p1_v)
            pltpu.sync_copy(g0b_hbm.at[pl.ds(tbase, _CCH)], g0_v)
            pltpu.sync_copy(g1b_hbm.at[pl.ds(tbase, _CCH)], g1_v)
            c0 = pltpu.async_copy(y_hbm.at[p0_v], a_v, sem)
            c1 = pltpu.async_copy(y_hbm.at[p1_v], b_v, sem)
            c0.wait()
            c1.wait()

            def row_body(i, carry):
                g0s = g0_v[i, :]
                g1s = g1_v[i, :]
                for j in range(H // 16):
                    sl = pl.ds(j * 16, 16)
                    s2 = pl.ds(H + j * 16, 16)
                    o_v[i, sl] = (g0s * (a_v[i, sl] + a_v[i, s2])
                                  + g1s * (b_v[i, sl] + b_v[i, s2]))
                return carry

            lax.fori_loop(0, _CCH, row_body, 0)
            pltpu.sync_copy(o_v, out_hbm.at[pl.ds(tbase, _CCH)])

    _sc_cache["k"] = (sc_dispatch, sc_combine)
    return _sc_cache["k"]


FH = F // 2


def _gemm_body(s_ref, x_ref, w1_ref, b1_ref, w2_ref, b2_ref, y_ref):
    half = pl.program_id(0)
    b = pl.program_id(1)

    @pl.when(b < s_ref[0])
    def _():
        x = x_ref[...]
        w1h = w1_ref[0].astype(jnp.bfloat16)
        h = jnp.dot(x, w1h, preferred_element_type=jnp.float32)
        h = jax.nn.gelu(h + b1_ref[0])
        w2h = w2_ref[0].astype(jnp.bfloat16)
        y = jnp.dot(h.astype(jnp.bfloat16), w2h,
                    preferred_element_type=jnp.float32)
        y = y + jnp.where(half == 0, 1.0, 0.0) * b2_ref[0]
        y_ref[...] = y


def _gemm(scalars, x_sorted, w1, b1r, w2, b2r):
    grid_spec = pltpu.PrefetchScalarGridSpec(
        num_scalar_prefetch=1,
        grid=(2, NB),
        in_specs=[
            pl.BlockSpec((BT, H),
                         lambda hf, b, s: (jnp.minimum(b, s[0] - 1), 0)),
            pl.BlockSpec((1, H, FH), lambda hf, b, s: (s[1 + b], 0, hf)),
            pl.BlockSpec((1, 1, FH), lambda hf, b, s: (s[1 + b], 0, hf)),
            pl.BlockSpec((1, FH, H), lambda hf, b, s: (s[1 + b], hf, 0)),
            pl.BlockSpec((1, 1, H), lambda hf, b, s: (s[1 + b], 0, 0)),
        ],
        out_specs=pl.BlockSpec(
            (BT, H), lambda hf, b, s: (jnp.minimum(b, s[0] - 1), hf)),
    )
    return pl.pallas_call(
        _gemm_body,
        grid_spec=grid_spec,
        out_shape=jax.ShapeDtypeStruct((P, 2 * H), jnp.float32),
    )(scalars, x_sorted, w1, b1r, w2, b2r)


def kernel(hidden_states, router_logits, w1, b1, w2, b2):
    pos, g0b, g1b, meta = _routing(router_logits)

    nb = meta[1, :].astype(jnp.int32)
    total = jnp.sum(nb)
    cum = jnp.cumsum(nb)
    be = jnp.searchsorted(cum, jnp.minimum(jnp.arange(NB), total - 1),
                          side="right").astype(jnp.int32)
    scalars = jnp.concatenate([total[None], be])

    sc_dispatch, sc_combine = _sc_kernels()
    pos0 = pos[:, 0]
    pos1 = pos[:, 1]
    xbi = jax.lax.bitcast_convert_type(
        hidden_states.astype(jnp.bfloat16).reshape(T, H // 2, 2), jnp.int32)
    x_sorted_i = sc_dispatch(pos0, pos1, xbi)
    x_sorted = jax.lax.bitcast_convert_type(
        x_sorted_i, jnp.bfloat16).reshape(P, H)

    b1r = b1.reshape(E, 1, F)
    b2r = b2.reshape(E, 1, H)
    y = _gemm(scalars, x_sorted, w1, b1r, w2, b2r)

    out = sc_combine(pos0, pos1, g0b, g1b, y)
    return out


# speedup vs baseline: 1.4982x; 1.4982x over previous
import functools

import jax
import jax.numpy as jnp
from jax import lax
from jax.experimental import pallas as pl
from jax.experimental.pallas import tpu as pltpu
from jax.experimental.pallas import tpu_sc as plsc

E = 8
K = 2
H = 1024
F = 4096
T = 2048
BT = 128
NB = (T * K) // BT + E
P = NB * BT
A = T * K

NC = 2
NS = 16
NW = NC * NS


def _routing_body(logits_ref, pos_ref, g0b_ref, g1b_ref, meta_ref, oh_ref,
                  pre_ref):
    lg = logits_ref[...]
    lane = lax.broadcasted_iota(jnp.int32, (T, E), 1)
    m1 = jnp.max(lg, axis=1, keepdims=True)
    i1 = jnp.min(jnp.where(lg == m1, lane, E), axis=1, keepdims=True)
    lg2 = jnp.where(lane == i1, -jnp.inf, lg)
    m2 = jnp.max(lg2, axis=1, keepdims=True)
    i2 = jnp.min(jnp.where(lg2 == m2, lane, E), axis=1, keepdims=True)
    t = jnp.exp(m2 - m1)
    g1 = 1.0 / (1.0 + t)
    g2 = t / (1.0 + t)

    sel1 = (lane == i1)
    sel2 = (lane == i2)
    oh = jnp.where(sel1 | sel2, 1.0, 0.0)
    oh_ref[...] = oh

    C = 512
    r = lax.broadcasted_iota(jnp.int32, (C, C), 0)
    c = lax.broadcasted_iota(jnp.int32, (C, C), 1)
    tri = jnp.where(r > c, 1.0, 0.0).astype(jnp.bfloat16)

    def chunk(ci, carry):
        ohc = oh_ref[pl.ds(ci * C, C), :]
        excl = jnp.dot(tri, ohc.astype(jnp.bfloat16),
                       preferred_element_type=jnp.float32)
        pre_ref[pl.ds(ci * C, C), :] = excl + carry
        return carry + jnp.sum(ohc, axis=0, keepdims=True)

    counts = lax.fori_loop(0, T // C, chunk, jnp.zeros((1, E), jnp.float32))

    nb = jnp.floor((counts + (BT - 1)) * (1.0 / BT))
    ur = lax.broadcasted_iota(jnp.int32, (E, E), 0)
    uc = lax.broadcasted_iota(jnp.int32, (E, E), 1)
    umat = jnp.where(ur < uc, 1.0, 0.0)
    pstart = jnp.dot(nb, umat, preferred_element_type=jnp.float32) * BT

    pre = pre_ref[...] + pstart
    pos1 = jnp.sum(jnp.where(sel1, pre, 0.0), axis=1, keepdims=True)
    pos2 = jnp.sum(jnp.where(sel2, pre, 0.0), axis=1, keepdims=True)
    pos_ref[...] = jnp.concatenate([pos1, pos2], axis=1).astype(jnp.int32)
    g0b_ref[...] = jnp.broadcast_to(g1, (T, 16))
    g1b_ref[...] = jnp.broadcast_to(g2, (T, 16))

    meta = jnp.concatenate([counts, nb, jnp.zeros((E - 2, E), jnp.float32)],
                           axis=0)
    meta_ref[...] = meta


def _routing(router_logits):
    return pl.pallas_call(
        _routing_body,
        out_shape=[
            jax.ShapeDtypeStruct((T, K), jnp.int32),
            jax.ShapeDtypeStruct((T, 16), jnp.float32),
            jax.ShapeDtypeStruct((T, 16), jnp.float32),
            jax.ShapeDtypeStruct((E, E), jnp.float32),
        ],
        in_specs=[pl.BlockSpec((T, E), lambda: (0, 0))],
        out_specs=[
            pl.BlockSpec((T, K), lambda: (0, 0)),
            pl.BlockSpec((T, 16), lambda: (0, 0)),
            pl.BlockSpec((T, 16), lambda: (0, 0)),
            pl.BlockSpec((E, E), lambda: (0, 0)),
        ],
        scratch_shapes=[
            pltpu.VMEM((T, E), jnp.float32),
            pltpu.VMEM((T, E), jnp.float32),
        ],
    )(router_logits)


_APW = A // NW
_RPW = P // NW
_GCH = _RPW // 2
_TPW = T // NW
_CCH = _TPW // 2

_sc_cache = {}


def _sc_kernels():
    if "k" in _sc_cache:
        return _sc_cache["k"]
    mesh = plsc.VectorSubcoreMesh(core_axis_name="c", subcore_axis_name="s")

    @functools.partial(
        pl.kernel, mesh=mesh,
        out_type=jax.ShapeDtypeStruct((P, H), jnp.float32),
        scratch_types=[
            pltpu.VMEM((_TPW,), jnp.int32),
            pltpu.VMEM((_TPW,), jnp.int32),
            pltpu.VMEM((_TPW, H), jnp.float32),
            pltpu.SemaphoreType.DMA,
        ],
    )
    def sc_dispatch(pos0_hbm, pos1_hbm, x_hbm, xs_out, p0_v, p1_v, rows_v,
                    sem):
        wid = lax.axis_index("s") * NC + lax.axis_index("c")
        base = wid * _TPW
        pltpu.sync_copy(x_hbm.at[pl.ds(base, _TPW)], rows_v)
        pltpu.sync_copy(pos0_hbm.at[pl.ds(base, _TPW)], p0_v)
        pltpu.sync_copy(pos1_hbm.at[pl.ds(base, _TPW)], p1_v)
        c0 = pltpu.async_copy(rows_v, xs_out.at[p0_v], sem)
        c1 = pltpu.async_copy(rows_v, xs_out.at[p1_v], sem)
        c0.wait()
        c1.wait()

    @functools.partial(
        pl.kernel, mesh=mesh,
        out_type=jax.ShapeDtypeStruct((T, H), jnp.float32),
        scratch_types=[
            pltpu.VMEM((_CCH,), jnp.int32),
            pltpu.VMEM((_CCH,), jnp.int32),
            pltpu.VMEM((_CCH,), jnp.int32),
            pltpu.VMEM((_CCH, 16), jnp.float32),
            pltpu.VMEM((_CCH, 16), jnp.float32),
            pltpu.VMEM((_CCH, H), jnp.float32),
            pltpu.VMEM((_CCH, H), jnp.float32),
            pltpu.VMEM((_CCH, H), jnp.float32),
            pltpu.SemaphoreType.DMA,
        ],
    )
    def sc_combine(pos0_hbm, pos1_hbm, g0b_hbm, g1b_hbm, y_hbm, out_hbm,
                   p0_v, p1_v, p2_v, g0_v, g1_v, a_v, b_v, c_v, sem):
        wid = lax.axis_index("s") * NC + lax.axis_index("c")
        for ch in range(2):
            tbase = wid * _TPW + ch * _CCH
            pltpu.sync_copy(pos0_hbm.at[pl.ds(tbase, _CCH)], p0_v)
            pltpu.sync_copy(pos1_hbm.at[pl.ds(tbase, _CCH)], p1_v)
            pltpu.sync_copy(g0b_hbm.at[pl.ds(tbase, _CCH)], g0_v)
            pltpu.sync_copy(g1b_hbm.at[pl.ds(tbase, _CCH)], g1_v)
            for j in range(_CCH // 16):
                sl = pl.ds(j * 16, 16)
                p2_v[sl] = p0_v[sl] + P
            c0 = pltpu.async_copy(y_hbm.at[p0_v], a_v, sem)
            c1 = pltpu.async_copy(y_hbm.at[p2_v], b_v, sem)
            c0.wait()
            c1.wait()

            def row_body1(i, carry):
                g0s = g0_v[i, :]
                for j in range(H // 16):
                    sl = pl.ds(j * 16, 16)
                    a_v[i, sl] = g0s * (a_v[i, sl] + b_v[i, sl])
                return carry

            lax.fori_loop(0, _CCH, row_body1, 0)

            for j in range(_CCH // 16):
                sl = pl.ds(j * 16, 16)
                p2_v[sl] = p1_v[sl] + P
            c2 = pltpu.async_copy(y_hbm.at[p1_v], b_v, sem)
            c3 = pltpu.async_copy(y_hbm.at[p2_v], c_v, sem)
            c2.wait()
            c3.wait()

            def row_body2(i, carry):
                g1s = g1_v[i, :]
                for j in range(H // 16):
                    sl = pl.ds(j * 16, 16)
                    a_v[i, sl] = a_v[i, sl] + g1s * (b_v[i, sl] + c_v[i, sl])
                return carry

            lax.fori_loop(0, _CCH, row_body2, 0)
            pltpu.sync_copy(a_v, out_hbm.at[pl.ds(tbase, _CCH)])

    _sc_cache["k"] = (sc_dispatch, sc_combine)
    return _sc_cache["k"]


FH = F // 2


def _gemm_body(s_ref, x_ref, w1_ref, b1_ref, w2_ref, b2_ref, y_ref):
    half = pl.program_id(0)
    b = pl.program_id(1)

    @pl.when(b < s_ref[0])
    def _():
        x = x_ref[...].astype(jnp.bfloat16)
        w1h = w1_ref[0].astype(jnp.bfloat16)
        h = jnp.dot(x, w1h, preferred_element_type=jnp.float32)
        h = jax.nn.gelu(h + b1_ref[0])
        w2h = w2_ref[0].astype(jnp.bfloat16)
        y = jnp.dot(h.astype(jnp.bfloat16), w2h,
                    preferred_element_type=jnp.float32)
        y = y + jnp.where(half == 0, 1.0, 0.0) * b2_ref[0]
        y_ref[0] = y


def _gemm(scalars, x_sorted, w1, b1r, w2, b2r):
    grid_spec = pltpu.PrefetchScalarGridSpec(
        num_scalar_prefetch=1,
        grid=(2, NB),
        in_specs=[
            pl.BlockSpec((BT, H),
                         lambda hf, b, s: (jnp.minimum(b, s[0] - 1), 0)),
            pl.BlockSpec((1, H, FH), lambda hf, b, s: (s[1 + b], 0, hf)),
            pl.BlockSpec((1, 1, FH), lambda hf, b, s: (s[1 + b], 0, hf)),
            pl.BlockSpec((1, FH, H), lambda hf, b, s: (s[1 + b], hf, 0)),
            pl.BlockSpec((1, 1, H), lambda hf, b, s: (s[1 + b], 0, 0)),
        ],
        out_specs=pl.BlockSpec(
            (1, BT, H), lambda hf, b, s: (hf, jnp.minimum(b, s[0] - 1), 0)),
    )
    return pl.pallas_call(
        _gemm_body,
        grid_spec=grid_spec,
        out_shape=jax.ShapeDtypeStruct((2, P, H), jnp.float32),
    )(scalars, x_sorted, w1, b1r, w2, b2r)


def kernel(hidden_states, router_logits, w1, b1, w2, b2):
    pos, g0b, g1b, meta = _routing(router_logits)

    nb = meta[1, :].astype(jnp.int32)
    total = jnp.sum(nb)
    cum = jnp.cumsum(nb)
    be = jnp.searchsorted(cum, jnp.minimum(jnp.arange(NB), total - 1),
                          side="right").astype(jnp.int32)
    scalars = jnp.concatenate([total[None], be])

    sc_dispatch, sc_combine = _sc_kernels()
    pos0 = pos[:, 0]
    pos1 = pos[:, 1]
    x_sorted = sc_dispatch(pos0, pos1, hidden_states)

    b1r = b1.reshape(E, 1, F)
    b2r = b2.reshape(E, 1, H)
    y = _gemm(scalars, x_sorted, w1, b1r, w2, b2r)

    out = sc_combine(pos0, pos1, g0b, g1b, y.reshape(2 * P, H))
    return out


# speedup vs baseline: 1.7309x; 1.1553x over previous
import functools

import jax
import jax.numpy as jnp
from jax import lax
from jax.experimental import pallas as pl
from jax.experimental.pallas import tpu as pltpu
from jax.experimental.pallas import tpu_sc as plsc

E = 8
K = 2
H = 1024
F = 4096
T = 2048
BT = 256
NB = (T * K) // BT + E
P = NB * BT
A = T * K

NC = 2
NS = 16
NW = NC * NS


def _routing_body(logits_ref, pos_ref, g0b_ref, g1b_ref, meta_ref, oh_ref,
                  pre_ref):
    lg = logits_ref[...]
    lane = lax.broadcasted_iota(jnp.int32, (T, E), 1)
    m1 = jnp.max(lg, axis=1, keepdims=True)
    i1 = jnp.min(jnp.where(lg == m1, lane, E), axis=1, keepdims=True)
    lg2 = jnp.where(lane == i1, -jnp.inf, lg)
    m2 = jnp.max(lg2, axis=1, keepdims=True)
    i2 = jnp.min(jnp.where(lg2 == m2, lane, E), axis=1, keepdims=True)
    t = jnp.exp(m2 - m1)
    g1 = 1.0 / (1.0 + t)
    g2 = t / (1.0 + t)

    sel1 = (lane == i1)
    sel2 = (lane == i2)
    oh = jnp.where(sel1 | sel2, 1.0, 0.0)
    oh_ref[...] = oh

    C = 512
    r = lax.broadcasted_iota(jnp.int32, (C, C), 0)
    c = lax.broadcasted_iota(jnp.int32, (C, C), 1)
    tri = jnp.where(r > c, 1.0, 0.0).astype(jnp.bfloat16)

    def chunk(ci, carry):
        ohc = oh_ref[pl.ds(ci * C, C), :]
        excl = jnp.dot(tri, ohc.astype(jnp.bfloat16),
                       preferred_element_type=jnp.float32)
        pre_ref[pl.ds(ci * C, C), :] = excl + carry
        return carry + jnp.sum(ohc, axis=0, keepdims=True)

    counts = lax.fori_loop(0, T // C, chunk, jnp.zeros((1, E), jnp.float32))

    nb = jnp.floor((counts + (BT - 1)) * (1.0 / BT))
    ur = lax.broadcasted_iota(jnp.int32, (E, E), 0)
    uc = lax.broadcasted_iota(jnp.int32, (E, E), 1)
    umat = jnp.where(ur < uc, 1.0, 0.0)
    pstart = jnp.dot(nb, umat, preferred_element_type=jnp.float32) * BT

    pre = pre_ref[...] + pstart
    pos1 = jnp.sum(jnp.where(sel1, pre, 0.0), axis=1, keepdims=True)
    pos2 = jnp.sum(jnp.where(sel2, pre, 0.0), axis=1, keepdims=True)
    pos_ref[...] = jnp.concatenate([pos1, pos2], axis=1).astype(jnp.int32)
    g0b_ref[...] = jnp.broadcast_to(g1, (T, 16))
    g1b_ref[...] = jnp.broadcast_to(g2, (T, 16))

    meta = jnp.concatenate([counts, nb, jnp.zeros((E - 2, E), jnp.float32)],
                           axis=0)
    meta_ref[...] = meta


def _routing(router_logits):
    return pl.pallas_call(
        _routing_body,
        out_shape=[
            jax.ShapeDtypeStruct((T, K), jnp.int32),
            jax.ShapeDtypeStruct((T, 16), jnp.float32),
            jax.ShapeDtypeStruct((T, 16), jnp.float32),
            jax.ShapeDtypeStruct((E, E), jnp.float32),
        ],
        in_specs=[pl.BlockSpec((T, E), lambda: (0, 0))],
        out_specs=[
            pl.BlockSpec((T, K), lambda: (0, 0)),
            pl.BlockSpec((T, 16), lambda: (0, 0)),
            pl.BlockSpec((T, 16), lambda: (0, 0)),
            pl.BlockSpec((E, E), lambda: (0, 0)),
        ],
        scratch_shapes=[
            pltpu.VMEM((T, E), jnp.float32),
            pltpu.VMEM((T, E), jnp.float32),
        ],
    )(router_logits)


_APW = A // NW
_RPW = P // NW
_GCH = _RPW // 2
_TPW = T // NW
_CCH = _TPW // 2

_sc_cache = {}


def _sc_kernels():
    if "k" in _sc_cache:
        return _sc_cache["k"]
    mesh = plsc.VectorSubcoreMesh(core_axis_name="c", subcore_axis_name="s")

    @functools.partial(
        pl.kernel, mesh=mesh,
        out_type=jax.ShapeDtypeStruct((P, H), jnp.float32),
        scratch_types=[
            pltpu.VMEM((_TPW,), jnp.int32),
            pltpu.VMEM((_TPW,), jnp.int32),
            pltpu.VMEM((_TPW, H), jnp.float32),
            pltpu.SemaphoreType.DMA,
        ],
    )
    def sc_dispatch(pos0_hbm, pos1_hbm, x_hbm, xs_out, p0_v, p1_v, rows_v,
                    sem):
        wid = lax.axis_index("s") * NC + lax.axis_index("c")
        base = wid * _TPW
        pltpu.sync_copy(x_hbm.at[pl.ds(base, _TPW)], rows_v)
        pltpu.sync_copy(pos0_hbm.at[pl.ds(base, _TPW)], p0_v)
        pltpu.sync_copy(pos1_hbm.at[pl.ds(base, _TPW)], p1_v)
        c0 = pltpu.async_copy(rows_v, xs_out.at[p0_v], sem)
        c1 = pltpu.async_copy(rows_v, xs_out.at[p1_v], sem)
        c0.wait()
        c1.wait()

    @functools.partial(
        pl.kernel, mesh=mesh,
        out_type=jax.ShapeDtypeStruct((T, H), jnp.float32),
        scratch_types=[
            pltpu.VMEM((_CCH,), jnp.int32),
            pltpu.VMEM((_CCH,), jnp.int32),
            pltpu.VMEM((_CCH,), jnp.int32),
            pltpu.VMEM((_CCH, 16), jnp.float32),
            pltpu.VMEM((_CCH, 16), jnp.float32),
            pltpu.VMEM((_CCH, H), jnp.float32),
            pltpu.VMEM((_CCH, H), jnp.float32),
            pltpu.VMEM((_CCH, H), jnp.float32),
            pltpu.SemaphoreType.DMA,
        ],
    )
    def sc_combine(pos0_hbm, pos1_hbm, g0b_hbm, g1b_hbm, y_hbm, out_hbm,
                   p0_v, p1_v, p2_v, g0_v, g1_v, a_v, b_v, c_v, sem):
        wid = lax.axis_index("s") * NC + lax.axis_index("c")
        for ch in range(2):
            tbase = wid * _TPW + ch * _CCH
            pltpu.sync_copy(pos0_hbm.at[pl.ds(tbase, _CCH)], p0_v)
            pltpu.sync_copy(pos1_hbm.at[pl.ds(tbase, _CCH)], p1_v)
            pltpu.sync_copy(g0b_hbm.at[pl.ds(tbase, _CCH)], g0_v)
            pltpu.sync_copy(g1b_hbm.at[pl.ds(tbase, _CCH)], g1_v)
            for j in range(_CCH // 16):
                sl = pl.ds(j * 16, 16)
                p2_v[sl] = p0_v[sl] + P
            c0 = pltpu.async_copy(y_hbm.at[p0_v], a_v, sem)
            c1 = pltpu.async_copy(y_hbm.at[p2_v], b_v, sem)
            c0.wait()
            c1.wait()

            def row_body1(i, carry):
                g0s = g0_v[i, :]
                for j in range(H // 16):
                    sl = pl.ds(j * 16, 16)
                    a_v[i, sl] = g0s * (a_v[i, sl] + b_v[i, sl])
                return carry

            lax.fori_loop(0, _CCH, row_body1, 0)

            for j in range(_CCH // 16):
                sl = pl.ds(j * 16, 16)
                p2_v[sl] = p1_v[sl] + P
            c2 = pltpu.async_copy(y_hbm.at[p1_v], b_v, sem)
            c3 = pltpu.async_copy(y_hbm.at[p2_v], c_v, sem)
            c2.wait()
            c3.wait()

            def row_body2(i, carry):
                g1s = g1_v[i, :]
                for j in range(H // 16):
                    sl = pl.ds(j * 16, 16)
                    a_v[i, sl] = a_v[i, sl] + g1s * (b_v[i, sl] + c_v[i, sl])
                return carry

            lax.fori_loop(0, _CCH, row_body2, 0)
            pltpu.sync_copy(a_v, out_hbm.at[pl.ds(tbase, _CCH)])

    _sc_cache["k"] = (sc_dispatch, sc_combine)
    return _sc_cache["k"]


FH = F // 2


def _gemm_body(s_ref, x_ref, w1_ref, b1_ref, w2_ref, b2_ref, y_ref):
    half = pl.program_id(0)
    b = pl.program_id(1)

    @pl.when(b < s_ref[0])
    def _():
        x = x_ref[...].astype(jnp.bfloat16)
        w1h = w1_ref[0].astype(jnp.bfloat16)
        h = jnp.dot(x, w1h, preferred_element_type=jnp.float32)
        h = jax.nn.gelu(h + b1_ref[0])
        w2h = w2_ref[0].astype(jnp.bfloat16)
        y = jnp.dot(h.astype(jnp.bfloat16), w2h,
                    preferred_element_type=jnp.float32)
        y = y + jnp.where(half == 0, 1.0, 0.0) * b2_ref[0]
        y_ref[0] = y


def _gemm(scalars, x_sorted, w1, b1r, w2, b2r):
    grid_spec = pltpu.PrefetchScalarGridSpec(
        num_scalar_prefetch=1,
        grid=(2, NB),
        in_specs=[
            pl.BlockSpec((BT, H),
                         lambda hf, b, s: (jnp.minimum(b, s[0] - 1), 0)),
            pl.BlockSpec((1, H, FH), lambda hf, b, s: (s[1 + b], 0, hf)),
            pl.BlockSpec((1, 1, FH), lambda hf, b, s: (s[1 + b], 0, hf)),
            pl.BlockSpec((1, FH, H), lambda hf, b, s: (s[1 + b], hf, 0)),
            pl.BlockSpec((1, 1, H), lambda hf, b, s: (s[1 + b], 0, 0)),
        ],
        out_specs=pl.BlockSpec(
            (1, BT, H), lambda hf, b, s: (hf, jnp.minimum(b, s[0] - 1), 0)),
    )
    return pl.pallas_call(
        _gemm_body,
        grid_spec=grid_spec,
        out_shape=jax.ShapeDtypeStruct((2, P, H), jnp.float32),
    )(scalars, x_sorted, w1, b1r, w2, b2r)


def kernel(hidden_states, router_logits, w1, b1, w2, b2):
    pos, g0b, g1b, meta = _routing(router_logits)

    nb = meta[1, :].astype(jnp.int32)
    total = jnp.sum(nb)
    cum = jnp.cumsum(nb)
    be = jnp.searchsorted(cum, jnp.minimum(jnp.arange(NB), total - 1),
                          side="right").astype(jnp.int32)
    scalars = jnp.concatenate([total[None], be])

    sc_dispatch, sc_combine = _sc_kernels()
    pos0 = pos[:, 0]
    pos1 = pos[:, 1]
    x_sorted = sc_dispatch(pos0, pos1, hidden_states)

    b1r = b1.reshape(E, 1, F)
    b2r = b2.reshape(E, 1, H)
    y = _gemm(scalars, x_sorted, w1, b1r, w2, b2r)

    out = sc_combine(pos0, pos1, g0b, g1b, y.reshape(2 * P, H))
    return out
